# initial kernel scaffold (unmeasured)
import jax
import jax.numpy as jnp
from jax import lax
from jax.experimental import pallas as pl
from jax.experimental.pallas import tpu as pltpu

N_DEV = 32
N_TOK = 2048
D_MODEL = 512
D_FF = 1024
N_EXPERTS = 128
E_LOCAL = N_EXPERTS // N_DEV
CHUNK = N_TOK // N_DEV


def _allreduce_body(partial_ref, out_ref, comm_ref, send_ref,
                    send_sems, recv_sems, credit0, credit1):
    d = lax.axis_index("i")
    left = lax.rem(d + N_DEV - 1, N_DEV)
    right = lax.rem(d + 1, N_DEV)
    credits = (credit0, credit1)

    barrier_sem = pltpu.get_barrier_semaphore()
    for nbr in (left, right):
        pl.semaphore_signal(
            barrier_sem, inc=1,
            device_id=(nbr,), device_id_type=pl.DeviceIdType.MESH,
        )
    pl.semaphore_wait(barrier_sem, 2)

    def rows(c):
        return pl.ds(c * CHUNK, CHUNK)

    send_ref[0] = partial_ref[rows(d)]

    for s in range(2 * N_DEV - 2):
        slot = s % 2
        if s >= 2:
            pl.semaphore_wait(credits[slot], 1)

        rdma = pltpu.make_async_remote_copy(
            src_ref=send_ref.at[slot],
            dst_ref=comm_ref.at[slot],
            send_sem=send_sems.at[slot],
            recv_sem=recv_sems.at[slot],
            device_id=(right,),
            device_id_type=pl.DeviceIdType.MESH,
        )
        rdma.start()
        rdma.wait()

        if s < N_DEV - 1:
            recv_c = lax.rem(d + 2 * N_DEV - s - 1, N_DEV)
            acc = comm_ref[slot] + partial_ref[rows(recv_c)]
            send_ref[1 - slot] = acc
            if s == N_DEV - 2:
                out_ref[rows(lax.rem(d + 1, N_DEV))] = acc
        else:
            t = s - (N_DEV - 1)
            recv_c = lax.rem(d + 2 * N_DEV - t, N_DEV)
            out_ref[rows(recv_c)] = comm_ref[slot]
            if t < N_DEV - 2:
                send_ref[1 - slot] = comm_ref[slot]

        pl.semaphore_signal(
            credits[slot], inc=1,
            device_id=(left,), device_id_type=pl.DeviceIdType.MESH,
        )


def _ring_allreduce(partial):
    return pl.pallas_call(
        _allreduce_body,
        out_shape=jax.ShapeDtypeStruct((N_TOK, D_FF), jnp.bfloat16),
        in_specs=[pl.BlockSpec(memory_space=pltpu.VMEM)],
        out_specs=pl.BlockSpec(memory_space=pltpu.VMEM),
        scratch_shapes=[
            pltpu.VMEM((2, CHUNK, D_FF), jnp.bfloat16),
            pltpu.VMEM((2, CHUNK, D_FF), jnp.bfloat16),
            pltpu.SemaphoreType.DMA((2,)),
            pltpu.SemaphoreType.DMA((2,)),
            pltpu.SemaphoreType.REGULAR,
            pltpu.SemaphoreType.REGULAR,
        ],
        compiler_params=pltpu.CompilerParams(collective_id=0),
    )(partial)


def kernel(x, router_W, route_idx, expert_W):
    scores = x @ router_W
    sel = jnp.take_along_axis(scores, route_idx, axis=1)
    g = jax.nn.softmax(sel, axis=-1)

    d = lax.axis_index("i")
    e_ids = d * E_LOCAL + jnp.arange(E_LOCAL, dtype=jnp.int32)
    G = jnp.zeros((N_TOK, E_LOCAL), dtype=jnp.float32)
    for k in range(2):
        G = G + g[:, k:k + 1] * (route_idx[:, k:k + 1] == e_ids[None, :])

    xb = x.astype(jnp.bfloat16)
    wb = expert_W.astype(jnp.bfloat16)
    partial = jnp.zeros((N_TOK, D_FF), dtype=jnp.float32)
    for e in range(E_LOCAL):
        xe = (xb * G[:, e:e + 1].astype(jnp.bfloat16))
        partial = partial + jnp.dot(
            xe, wb[e], preferred_element_type=jnp.float32
        )
    partial = partial.astype(jnp.bfloat16)

    return _ring_allreduce(partial)


# baseline (device time: 239165 ns/iter reference)
import jax
import jax.numpy as jnp
from jax import lax
from jax.experimental import pallas as pl
from jax.experimental.pallas import tpu as pltpu

N_DEV = 32
N_TOK = 2048
D_MODEL = 512
D_FF = 1024
N_EXPERTS = 128
E_LOCAL = N_EXPERTS // N_DEV
CHUNK = N_TOK // N_DEV


N_STEPS = 2 * N_DEV - 2


def _allreduce_body(partial_ref, out_ref, comm_ref, send_sems, recv_sems):
    d = lax.axis_index("i")
    left = lax.rem(d + N_DEV - 1, N_DEV)
    right = lax.rem(d + 1, N_DEV)

    barrier_sem = pltpu.get_barrier_semaphore()
    for nbr in (left, right):
        pl.semaphore_signal(
            barrier_sem, inc=1,
            device_id=(nbr,), device_id_type=pl.DeviceIdType.MESH,
        )
    pl.semaphore_wait(barrier_sem, 2)

    def rows(c):
        return pl.ds(c * CHUNK, CHUNK)

    comm_ref[0] = partial_ref[rows(d)]

    for s in range(N_STEPS):
        rdma = pltpu.make_async_remote_copy(
            src_ref=comm_ref.at[s],
            dst_ref=comm_ref.at[s + 1],
            send_sem=send_sems.at[s],
            recv_sem=recv_sems.at[s],
            device_id=(right,),
            device_id_type=pl.DeviceIdType.MESH,
        )
        rdma.start()
        rdma.wait()

        if s < N_DEV - 1:
            recv_c = lax.rem(d + 2 * N_DEV - s - 1, N_DEV)
            comm_ref[s + 1] = comm_ref[s + 1] + partial_ref[rows(recv_c)]
            if s == N_DEV - 2:
                out_ref[rows(lax.rem(d + 1, N_DEV))] = comm_ref[s + 1]
        else:
            t = s - (N_DEV - 1)
            recv_c = lax.rem(d + 2 * N_DEV - t, N_DEV)
            out_ref[rows(recv_c)] = comm_ref[s + 1]


def _ring_allreduce(partial):
    return pl.pallas_call(
        _allreduce_body,
        out_shape=jax.ShapeDtypeStruct((N_TOK, D_FF), jnp.bfloat16),
        in_specs=[pl.BlockSpec(memory_space=pltpu.VMEM)],
        out_specs=pl.BlockSpec(memory_space=pltpu.VMEM),
        scratch_shapes=[
            pltpu.VMEM((N_STEPS + 1, CHUNK, D_FF), jnp.bfloat16),
            pltpu.SemaphoreType.DMA((N_STEPS,)),
            pltpu.SemaphoreType.DMA((N_STEPS,)),
        ],
        compiler_params=pltpu.CompilerParams(collective_id=0),
    )(partial)


def kernel(x, router_W, route_idx, expert_W):
    scores = x @ router_W
    sel = jnp.take_along_axis(scores, route_idx, axis=1)
    g = jax.nn.softmax(sel, axis=-1)

    d = lax.axis_index("i")
    e_ids = d * E_LOCAL + jnp.arange(E_LOCAL, dtype=jnp.int32)
    G = jnp.zeros((N_TOK, E_LOCAL), dtype=jnp.float32)
    for k in range(2):
        G = G + g[:, k:k + 1] * (route_idx[:, k:k + 1] == e_ids[None, :])

    xb = x.astype(jnp.bfloat16)
    wb = expert_W.astype(jnp.bfloat16)
    partial = jnp.zeros((N_TOK, D_FF), dtype=jnp.float32)
    for e in range(E_LOCAL):
        xe = (xb * G[:, e:e + 1].astype(jnp.bfloat16))
        partial = partial + jnp.dot(
            xe, wb[e], preferred_element_type=jnp.float32
        )
    partial = partial.astype(jnp.bfloat16)

    return _ring_allreduce(partial)


# device time: 128671 ns/iter; 1.8587x vs baseline; 1.8587x over previous
import jax
import jax.numpy as jnp
from jax import lax
from jax.experimental import pallas as pl
from jax.experimental.pallas import tpu as pltpu

N_DEV = 32
N_TOK = 2048
D_MODEL = 512
D_FF = 1024
N_EXPERTS = 128
E_LOCAL = N_EXPERTS // N_DEV
CHUNK = N_TOK // N_DEV


def _moe_allreduce_body(xb_ref, gb_ref, wb_ref, out_ref, rs_ref, red_ref,
                        snd_ref, s1_sems, r1_sems, s2_sems, r2_sems):
    d = lax.axis_index("i")

    def rows(c):
        return pl.ds(c * CHUNK, CHUNK)

    barrier_sem = pltpu.get_barrier_semaphore()
    peers = [lax.rem(d + o, N_DEV) for o in range(1, N_DEV)]
    for p in peers:
        pl.semaphore_signal(
            barrier_sem, inc=1,
            device_id=(p,), device_id_type=pl.DeviceIdType.MESH,
        )
    pl.semaphore_wait(barrier_sem, N_DEV - 1)

    BLK = 4
    CPB = N_DEV // BLK
    ROWS_PB = CPB * CHUNK

    def send_chunk(c):
        rdma = pltpu.make_async_remote_copy(
            src_ref=snd_ref.at[rows(c)],
            dst_ref=rs_ref.at[d],
            send_sem=s1_sems.at[c],
            recv_sem=r1_sems.at[d],
            device_id=(c,),
            device_id_type=pl.DeviceIdType.MESH,
        )
        return rdma

    d_blk = lax.div(d, CPB)
    d_in = lax.rem(d, CPB)
    for bi in range(BLK):
        b = lax.rem(d_blk + 1 + bi, BLK)
        blk_rows = pl.ds(b * ROWS_PB, ROWS_PB)
        xc = xb_ref[blk_rows, :]
        gc = gb_ref[blk_rows, :]
        acc = jnp.dot(
            xc * gc[:, 0:1], wb_ref[0],
            preferred_element_type=jnp.float32,
        )
        for e in range(1, E_LOCAL):
            acc = acc + jnp.dot(
                xc * gc[:, e:e + 1], wb_ref[e],
                preferred_element_type=jnp.float32,
            )
        snd_ref[blk_rows, :] = acc.astype(jnp.bfloat16)
        for k in range(CPB):
            c = b * CPB + lax.rem(d_in + 1 + k, CPB)
            @pl.when(c != d)
            def _(c=c):
                send_chunk(c).start()

    own = snd_ref[rows(d), :].astype(jnp.float32)

    acc = own
    rev = list(range(N_DEV - 1, 0, -1))
    for i in range(0, len(rev), 8):
        group = rev[i:i + 8]
        for o in group:
            p = peers[o - 1]
            recv = pltpu.make_async_remote_copy(
                src_ref=red_ref,
                dst_ref=rs_ref.at[p],
                send_sem=s1_sems.at[p],
                recv_sem=r1_sems.at[p],
                device_id=(p,),
                device_id_type=pl.DeviceIdType.MESH,
            )
            recv.wait_recv()
        got = [rs_ref[peers[o - 1]] for o in group]
        while len(got) > 1:
            got = [a + b for a, b in zip(got[::2], got[1::2])] + (
                [got[-1]] if len(got) % 2 else [])
        acc = acc + got[0].astype(jnp.float32)
    red = acc.astype(jnp.bfloat16)
    red_ref[...] = red
    out_ref[rows(d)] = red

    sends2 = []
    for p in peers:
        rdma = pltpu.make_async_remote_copy(
            src_ref=red_ref,
            dst_ref=out_ref.at[rows(d)],
            send_sem=s2_sems.at[p],
            recv_sem=r2_sems.at[d],
            device_id=(p,),
            device_id_type=pl.DeviceIdType.MESH,
        )
        rdma.start()
        sends2.append(rdma)

    for p in reversed(peers):
        recv = pltpu.make_async_remote_copy(
            src_ref=red_ref,
            dst_ref=out_ref.at[rows(p)],
            send_sem=s2_sems.at[p],
            recv_sem=r2_sems.at[p],
            device_id=(p,),
            device_id_type=pl.DeviceIdType.MESH,
        )
        recv.wait_recv()

    for c in range(N_DEV):
        @pl.when(c != d)
        def _(c=c):
            send_chunk(c).wait_send()
    for rdma in sends2:
        rdma.wait_send()


def _moe_allreduce(xb, gb, wb):
    return pl.pallas_call(
        _moe_allreduce_body,
        out_shape=jax.ShapeDtypeStruct((N_TOK, D_FF), jnp.bfloat16),
        in_specs=[
            pl.BlockSpec(memory_space=pltpu.VMEM),
            pl.BlockSpec(memory_space=pltpu.VMEM),
            pl.BlockSpec(memory_space=pltpu.VMEM),
        ],
        out_specs=pl.BlockSpec(memory_space=pltpu.VMEM),
        scratch_shapes=[
            pltpu.VMEM((N_DEV, CHUNK, D_FF), jnp.bfloat16),
            pltpu.VMEM((CHUNK, D_FF), jnp.bfloat16),
            pltpu.VMEM((N_TOK, D_FF), jnp.bfloat16),
            pltpu.SemaphoreType.DMA((N_DEV,)),
            pltpu.SemaphoreType.DMA((N_DEV,)),
            pltpu.SemaphoreType.DMA((N_DEV,)),
            pltpu.SemaphoreType.DMA((N_DEV,)),
        ],
        compiler_params=pltpu.CompilerParams(collective_id=0),
    )(xb, gb, wb)


def kernel(x, router_W, route_idx, expert_W):
    scores = x @ router_W
    e_all = jnp.arange(N_EXPERTS, dtype=jnp.int32)
    m0 = (route_idx[:, 0:1] == e_all[None, :])
    m1 = (route_idx[:, 1:2] == e_all[None, :])
    s0 = jnp.sum(jnp.where(m0, scores, 0.0), axis=1)
    s1 = jnp.sum(jnp.where(m1, scores, 0.0), axis=1)
    mx = jnp.maximum(s0, s1)
    g0 = jnp.exp(s0 - mx)
    g1 = jnp.exp(s1 - mx)
    inv = 1.0 / (g0 + g1)

    d = lax.axis_index("i")
    e_ids = d * E_LOCAL + jnp.arange(E_LOCAL, dtype=jnp.int32)
    loc0 = (route_idx[:, 0:1] == e_ids[None, :])
    loc1 = (route_idx[:, 1:2] == e_ids[None, :])
    G = (g0 * inv)[:, None] * loc0 + (g1 * inv)[:, None] * loc1

    xb = x.astype(jnp.bfloat16)
    Gb = G.astype(jnp.bfloat16)
    wb = expert_W.astype(jnp.bfloat16)
    return _moe_allreduce(xb, Gb, wb)
